# baseline (speedup 1.0000x reference)
import functools
import math

import jax
import jax.numpy as jnp
from jax import lax
from jax.experimental import pallas as pl
from jax.experimental.pallas import tpu as pltpu
from jax.experimental.pallas import tpu_sc as plsc

N = 1 << 20
NSUB = 16
CHUNK = N // NSUB
VREGS = CHUNK // 16
RADIX = 256
NBINS = NSUB * RADIX

_LAM = 0.1
_C = 1.0 - math.exp(-_LAM)

_SIGN = 0x80000000


def _monotonize_i32(b):
    t = jax.lax.shift_right_logical(b, jnp.int32(31))
    mask = jnp.int32(-(1 << 31)) | (jnp.int32(0) - t)
    return b ^ mask


def _sc_body(x_hbm, u_hbm, outx_hbm, outu_hbm, b1x_hbm, b2x_hbm,
             b1u_hbm, b2u_hbm,
             chunk_v, hist_v, off2_v, tot_v, pa_v, colsum_v, gridv_v,
             grid_sh, sem):
    c = lax.axis_index("c")
    s = lax.axis_index("s")
    base = s * CHUNK
    lane = lax.iota(jnp.int32, 16)
    lane_sh = lane << jnp.int32(8)
    ones_i = jnp.ones((16,), jnp.int32)
    zeros_i = jnp.zeros((16,), jnp.int32)

    def zero_ref(ref, nv):
        def zb(i, _):
            ref[pl.ds(i * 16, 16)] = zeros_i
            return 0
        lax.fori_loop(0, nv, zb, 0)

    def pipeline(in_hbm, out_hbm, b1, b2):
        stages = [(in_hbm, b1), (b1, b2), (b2, b1), (b1, out_hbm)]
        for p, (src, dst) in enumerate(stages):
            shift = jnp.int32(8 * p)
            first = p == 0
            last = p == 3

            pltpu.sync_copy(src.at[pl.ds(base, CHUNK)], chunk_v)

            zero_ref(hist_v, NBINS // 16)

            def h_body(v, _):
                k = chunk_v[pl.ds(v * 16, 16)]
                if first:
                    k = _monotonize_i32(k)
                    chunk_v[pl.ds(v * 16, 16)] = k
                d = jax.lax.shift_right_logical(k, shift) & jnp.int32(0xFF)
                bin_i = lane_sh | d
                plsc.addupdate_scatter(hist_v, [bin_i], ones_i)
                return 0
            lax.fori_loop(0, VREGS, h_body, 0)

            zero_ref(tot_v, RADIX // 16)

            def t_body(i, _):
                l = i // 16
                t = i % 16
                tot_v[pl.ds(t * 16, 16)] = (tot_v[pl.ds(t * 16, 16)]
                                            + hist_v[pl.ds(l * RADIX + t * 16, 16)])
                return 0
            lax.fori_loop(0, NBINS // 16, t_body, 0)

            def b_body(t, _):
                run = zeros_i
                for l in range(NSUB):
                    off2_v[pl.ds(l * RADIX + t * 16, 16)] = run
                    run = run + hist_v[pl.ds(l * RADIX + t * 16, 16)]
                return 0
            lax.fori_loop(0, RADIX // 16, b_body, 0)

            pltpu.sync_copy(tot_v, grid_sh.at[pl.ds(s * RADIX, RADIX)])
            plsc.subcore_barrier()
            pltpu.sync_copy(grid_sh, gridv_v)

            zero_ref(pa_v, RADIX // 16)

            def a_body(r, _):
                for t in range(RADIX // 16):
                    pa_v[pl.ds(t * 16, 16)] = (pa_v[pl.ds(t * 16, 16)]
                                               + gridv_v[pl.ds(r * RADIX + t * 16, 16)])
                return 0
            lax.fori_loop(0, s, a_body, 0)

            zero_ref(colsum_v, RADIX // 16)

            def c_body(r, _):
                for t in range(RADIX // 16):
                    colsum_v[pl.ds(t * 16, 16)] = (colsum_v[pl.ds(t * 16, 16)]
                                                   + gridv_v[pl.ds(r * RADIX + t * 16, 16)])
                return 0
            lax.fori_loop(0, NSUB, c_body, 0)

            carry = jnp.int32(0)
            for t in range(RADIX // 16):
                cs = colsum_v[pl.ds(t * 16, 16)]
                e = plsc.cumsum(cs)
                pa_v[pl.ds(t * 16, 16)] = (pa_v[pl.ds(t * 16, 16)]
                                           + (e - cs) + carry)
                carry = carry + jnp.sum(cs)

            def f_body(i, _):
                l = i // 16
                t = i % 16
                off2_v[pl.ds(l * RADIX + t * 16, 16)] = (
                    off2_v[pl.ds(l * RADIX + t * 16, 16)] + pa_v[pl.ds(t * 16, 16)])
                return 0
            lax.fori_loop(0, NBINS // 16, f_body, 0)

            GROUP = 8

            def drain_one():
                pltpu.make_async_copy(src.at[pl.ds(0, 16)],
                                      chunk_v.at[pl.ds(0, 16)], sem).wait()

            def s_body(g, _):
                for b in range(GROUP):
                    v = g * GROUP + b
                    k = chunk_v[pl.ds(v * 16, 16)]
                    d = jax.lax.shift_right_logical(k, shift) & jnp.int32(0xFF)
                    bin_i = lane_sh | d
                    q = plsc.load_gather(off2_v, [bin_i])
                    plsc.store_scatter(off2_v, [bin_i], q + 1)
                    if last:
                        pos = q
                    else:
                        sf = q >> jnp.int32(16)
                        lf = (q >> jnp.int32(12)) & jnp.int32(0xF)
                        vf = q & jnp.int32(0xFFF)
                        pos = (sf << jnp.int32(16)) | (vf << jnp.int32(4)) | lf
                    pltpu.async_copy(
                        chunk_v.at[pl.ds(v * 16, 16)], dst.at[pos], sem)

                @pl.when(g >= 2)
                def _():
                    for _b in range(GROUP):
                        drain_one()
                return 0
            lax.fori_loop(0, VREGS // GROUP, s_body, 0)
            for _b in range(2 * GROUP):
                drain_one()

            plsc.subcore_barrier()

    @pl.when(c == 0)
    def _():
        pipeline(x_hbm, outx_hbm, b1x_hbm, b2x_hbm)

    @pl.when(c == 1)
    def _():
        pipeline(u_hbm, outu_hbm, b1u_hbm, b2u_hbm)


def sc_sort_keys(x, u):
    mesh = plsc.VectorSubcoreMesh(core_axis_name="c", subcore_axis_name="s",
                                  num_cores=2, num_subcores=NSUB)
    f = pl.kernel(
        _sc_body,
        out_type=[jax.ShapeDtypeStruct((N,), jnp.int32)] * 6,
        mesh=mesh,
        compiler_params=pltpu.CompilerParams(needs_layout_passes=False),
        scratch_types=[
            pltpu.VMEM((CHUNK,), jnp.int32),
            pltpu.VMEM((NBINS,), jnp.int32),
            pltpu.VMEM((NBINS,), jnp.int32),
            pltpu.VMEM((RADIX,), jnp.int32),
            pltpu.VMEM((RADIX,), jnp.int32),
            pltpu.VMEM((RADIX,), jnp.int32),
            pltpu.VMEM((NBINS,), jnp.int32),
            pltpu.VMEM_SHARED((NBINS,), jnp.int32),
            pltpu.SemaphoreType.DMA,
        ],
    )
    return f(x, u)


LANES = 128


def _tc_loss_body(kx_ref, ku_ref, out_ref):
    def demonotonize(ki32):
        k = jax.lax.bitcast_convert_type(ki32[...], jnp.uint32)
        t = k >> jnp.uint32(31)
        mask = jnp.uint32(_SIGN) | (t - jnp.uint32(1))
        return jax.lax.bitcast_convert_type(k ^ mask, jnp.float32)

    xs = demonotonize(kx_ref)
    us = demonotonize(ku_ref)
    i_seq = -jnp.log(1.0 - _C * us) / _LAM
    d = i_seq - xs
    out_ref[...] = jnp.mean(d * d, keepdims=True)


def kernel(x, u):
    xi = jax.lax.bitcast_convert_type(x, jnp.int32)
    ui = jax.lax.bitcast_convert_type(u, jnp.int32)
    kx, ku = sc_sort_keys(xi, ui)[:2]
    R = N // LANES
    out = pl.pallas_call(
        _tc_loss_body,
        out_shape=jax.ShapeDtypeStruct((1, 1), jnp.float32),
    )(kx.reshape(R, LANES), ku.reshape(R, LANES))
    return out[0, 0]
